# baseline (speedup 1.0000x reference)
import jax
import jax.numpy as jnp
from jax.experimental import pallas as pl
from jax.experimental.pallas import tpu as pltpu


def _reduce_mlp_body(nnzf, vals_ref, rtw_ref, rtb_ref, w1t_ref, b1_ref,
                     w2_ref, b2_ref, att_ref, sc_ref, of_ref, acc_ref):
    j = pl.program_id(0)
    nb = pl.num_programs(0)

    @pl.when(j == 0)
    def _():
        acc_ref[...] = jnp.zeros_like(acc_ref)

    acc_ref[...] += vals_ref[...]

    @pl.when(j == nb - 1)
    def _():
        sums = jnp.sum(acc_ref[...], axis=1, keepdims=True)
        feats = rtw_ref[...] * sums + nnzf * rtb_ref[...]
        h = jnp.sum(feats * w1t_ref[...], axis=0, keepdims=True) + b1_ref[...]
        h = jnp.maximum(h, 0.0)
        logits = (jnp.sum(h * w2_ref[...], axis=1, keepdims=True)
                  + b2_ref[...])
        m = jnp.max(logits, axis=0, keepdims=True)
        e = jnp.exp(logits - m)
        att = e / jnp.sum(e, axis=0, keepdims=True)
        att_ref[...] = att
        sc_ref[...] = att * rtw_ref[...]
        of_ref[...] = att * rtb_ref[...]


def _scale_body_1d(R, vals_ref, sc_ref, of_ref, vout_ref):
    r = pl.program_id(1)
    for rr in range(R):
        @pl.when(r == rr)
        def _(rr=rr):
            vout_ref[...] = (vals_ref[rr, :] * sc_ref[rr, 0]
                             + of_ref[rr, 0])


def kernel(A_indices, A_values, rt_w, rt_b, W1, b1, W2, b2):
    R, NNZ = A_values.shape
    BN = min(65536, NNZ)
    nb = NNZ // BN
    H = W1.shape[0]

    idx_out = jnp.transpose(A_indices, (1, 0, 2))

    def body(*refs):
        _reduce_mlp_body(float(NNZ), *refs)

    small = lambda j: (0, 0)
    att_c, scale_c, off_c = pl.pallas_call(
        body,
        grid=(nb,),
        in_specs=[
            pl.BlockSpec((R, BN), lambda j: (0, j)),
            pl.BlockSpec((R, 1), small),
            pl.BlockSpec((R, 1), small),
            pl.BlockSpec((R, H), small),
            pl.BlockSpec((1, H), small),
            pl.BlockSpec((R, H), small),
            pl.BlockSpec((R, 1), small),
        ],
        out_specs=[pl.BlockSpec((R, 1), small)] * 3,
        out_shape=[jax.ShapeDtypeStruct((R, 1), jnp.float32)] * 3,
        scratch_shapes=[pltpu.VMEM((R, BN), jnp.float32)],
    )(A_values, rt_w.reshape(R, 1), rt_b.reshape(R, 1), W1.T,
      b1.reshape(1, H), W2, b2.reshape(R, 1))

    def scale_body(*refs):
        _scale_body_1d(R, *refs)

    vals_out = pl.pallas_call(
        scale_body,
        grid=(nb, R),
        in_specs=[
            pl.BlockSpec((R, BN), lambda j, r: (0, j)),
            pl.BlockSpec((R, 1), lambda j, r: (0, 0)),
            pl.BlockSpec((R, 1), lambda j, r: (0, 0)),
        ],
        out_specs=pl.BlockSpec((BN,), lambda j, r: (r * (NNZ // BN) + j,)),
        out_shape=jax.ShapeDtypeStruct((R * NNZ,), jnp.float32),
    )(A_values, scale_c, off_c)

    return (idx_out.reshape(2, R * NNZ), vals_out, att_c.reshape(R))


# speedup vs baseline: 1.0019x; 1.0019x over previous
import jax
import jax.numpy as jnp
from jax.experimental import pallas as pl
from jax.experimental.pallas import tpu as pltpu


def _reduce_mlp_body(nnzf, vals_ref, rtw_ref, rtb_ref, w1t_ref, b1_ref,
                     w2_ref, b2_ref, att_ref, sc_ref, of_ref, acc_ref):
    j = pl.program_id(0)
    nb = pl.num_programs(0)

    @pl.when(j == 0)
    def _():
        acc_ref[...] = jnp.zeros_like(acc_ref)

    acc_ref[...] += vals_ref[...]

    @pl.when(j == nb - 1)
    def _():
        sums = jnp.sum(acc_ref[...], axis=1, keepdims=True)
        feats = rtw_ref[...] * sums + nnzf * rtb_ref[...]
        h = jnp.sum(feats * w1t_ref[...], axis=0, keepdims=True) + b1_ref[...]
        h = jnp.maximum(h, 0.0)
        logits = (jnp.sum(h * w2_ref[...], axis=1, keepdims=True)
                  + b2_ref[...])
        m = jnp.max(logits, axis=0, keepdims=True)
        e = jnp.exp(logits - m)
        att = e / jnp.sum(e, axis=0, keepdims=True)
        att_ref[...] = att
        sc_ref[...] = att * rtw_ref[...]
        of_ref[...] = att * rtb_ref[...]


def _scale_body_1d(R, vals_ref, sc_ref, of_ref, vout_ref):
    r = pl.program_id(1)
    for rr in range(R):
        @pl.when(r == rr)
        def _(rr=rr):
            vout_ref[...] = (vals_ref[rr, :] * sc_ref[rr, 0]
                             + of_ref[rr, 0])


def kernel(A_indices, A_values, rt_w, rt_b, W1, b1, W2, b2):
    R, NNZ = A_values.shape
    BN = min(65536, NNZ)
    nb = NNZ // BN
    H = W1.shape[0]

    idx_out = jnp.transpose(A_indices, (1, 0, 2))

    def body(*refs):
        _reduce_mlp_body(float(NNZ), *refs)

    small = lambda j: (0, 0)
    att_c, scale_c, off_c = pl.pallas_call(
        body,
        grid=(nb,),
        in_specs=[
            pl.BlockSpec((R, BN), lambda j: (0, j)),
            pl.BlockSpec((R, 1), small),
            pl.BlockSpec((R, 1), small),
            pl.BlockSpec((R, H), small),
            pl.BlockSpec((1, H), small),
            pl.BlockSpec((R, H), small),
            pl.BlockSpec((R, 1), small),
        ],
        out_specs=[pl.BlockSpec((R, 1), small)] * 3,
        out_shape=[jax.ShapeDtypeStruct((R, 1), jnp.float32)] * 3,
        scratch_shapes=[pltpu.VMEM((R, BN), jnp.float32)],
        cost_estimate=pl.CostEstimate(
            flops=R * NNZ, transcendentals=64,
            bytes_accessed=4 * R * NNZ),
    )(A_values, rt_w.reshape(R, 1), rt_b.reshape(R, 1), W1.T,
      b1.reshape(1, H), W2, b2.reshape(R, 1))

    def scale_body(*refs):
        _scale_body_1d(R, *refs)

    vals_out = pl.pallas_call(
        scale_body,
        grid=(nb, R),
        in_specs=[
            pl.BlockSpec((R, BN), lambda j, r: (0, j)),
            pl.BlockSpec((R, 1), lambda j, r: (0, 0)),
            pl.BlockSpec((R, 1), lambda j, r: (0, 0)),
        ],
        out_specs=pl.BlockSpec((BN,), lambda j, r: (r * (NNZ // BN) + j,)),
        out_shape=jax.ShapeDtypeStruct((R * NNZ,), jnp.float32),
        cost_estimate=pl.CostEstimate(
            flops=2 * R * NNZ, transcendentals=0,
            bytes_accessed=8 * R * NNZ),
    )(A_values, scale_c, off_c)

    return (idx_out.reshape(2, R * NNZ), vals_out, att_c.reshape(R))


# speedup vs baseline: 1.4165x; 1.4137x over previous
import jax
import jax.numpy as jnp
from jax.experimental import pallas as pl
from jax.experimental.pallas import tpu as pltpu


def _reduce_mlp_body(nnzf, vals_ref, rtw_ref, rtb_ref, w1t_ref, b1_ref,
                     w2_ref, b2_ref, att_ref, sc_ref, of_ref, acc_ref):
    j = pl.program_id(0)
    nb = pl.num_programs(0)

    @pl.when(j == 0)
    def _():
        acc_ref[...] = jnp.zeros_like(acc_ref)

    acc_ref[...] += vals_ref[...]

    @pl.when(j == nb - 1)
    def _():
        sums = jnp.sum(acc_ref[...], axis=1, keepdims=True)
        feats = rtw_ref[...] * sums + nnzf * rtb_ref[...]
        h = jnp.sum(feats * w1t_ref[...], axis=0, keepdims=True) + b1_ref[...]
        h = jnp.maximum(h, 0.0)
        logits = (jnp.sum(h * w2_ref[...], axis=1, keepdims=True)
                  + b2_ref[...])
        m = jnp.max(logits, axis=0, keepdims=True)
        e = jnp.exp(logits - m)
        att = e / jnp.sum(e, axis=0, keepdims=True)
        att_ref[...] = att
        sc_ref[...] = att * rtw_ref[...]
        of_ref[...] = att * rtb_ref[...]


def _scale_dma_body(R, NNZ, BN, nb, vals_ref, sc_ref, of_ref, out_ref,
                    buf_ref, sems):
    j = pl.program_id(0)

    def desc(bb, rr):
        return pltpu.make_async_copy(
            buf_ref.at[bb, rr],
            out_ref.at[pl.ds(rr * NNZ + j * BN, BN)],
            sems.at[bb, rr])

    for bb in range(2):
        @pl.when(j % 2 == bb)
        def _(bb=bb):
            @pl.when(j >= 2)
            def _():
                for rr in range(R):
                    desc(bb, rr).wait()

            buf_ref[bb] = vals_ref[...] * sc_ref[...] + of_ref[...]
            for rr in range(R):
                desc(bb, rr).start()

            @pl.when(j == nb - 1)
            def _():
                for rr in range(R):
                    desc(bb, rr).wait()
                if nb > 1:
                    for rr in range(R):
                        desc(1 - bb, rr).wait()


def kernel(A_indices, A_values, rt_w, rt_b, W1, b1, W2, b2):
    R, NNZ = A_values.shape
    BN = min(65536, NNZ // 8)
    nb = NNZ // BN
    H = W1.shape[0]

    idx_out = jnp.transpose(A_indices, (1, 0, 2))

    def body(*refs):
        _reduce_mlp_body(float(NNZ), *refs)

    small = lambda j: (0, 0)
    att_c, scale_c, off_c = pl.pallas_call(
        body,
        grid=(nb,),
        in_specs=[
            pl.BlockSpec((R, BN), lambda j: (0, j)),
            pl.BlockSpec((R, 1), small),
            pl.BlockSpec((R, 1), small),
            pl.BlockSpec((R, H), small),
            pl.BlockSpec((1, H), small),
            pl.BlockSpec((R, H), small),
            pl.BlockSpec((R, 1), small),
        ],
        out_specs=[pl.BlockSpec((R, 1), small)] * 3,
        out_shape=[jax.ShapeDtypeStruct((R, 1), jnp.float32)] * 3,
        scratch_shapes=[pltpu.VMEM((R, BN), jnp.float32)],
        cost_estimate=pl.CostEstimate(
            flops=R * NNZ, transcendentals=64,
            bytes_accessed=4 * R * NNZ),
    )(A_values, rt_w.reshape(R, 1), rt_b.reshape(R, 1), W1.T,
      b1.reshape(1, H), W2, b2.reshape(R, 1))

    def scale_body(*refs):
        _scale_dma_body(R, NNZ, BN, nb, *refs)

    vals_out = pl.pallas_call(
        scale_body,
        grid=(nb,),
        in_specs=[
            pl.BlockSpec((R, BN), lambda j: (0, j)),
            pl.BlockSpec((R, 1), small),
            pl.BlockSpec((R, 1), small),
        ],
        out_specs=pl.BlockSpec(memory_space=pl.ANY),
        out_shape=jax.ShapeDtypeStruct((R * NNZ,), jnp.float32),
        scratch_shapes=[
            pltpu.VMEM((2, R, BN), jnp.float32),
            pltpu.SemaphoreType.DMA((2, R)),
        ],
    )(A_values, scale_c, off_c)

    return (idx_out.reshape(2, R * NNZ), vals_out, att_c.reshape(R))


# speedup vs baseline: 1.6658x; 1.1760x over previous
import jax
import jax.numpy as jnp
from jax.experimental import pallas as pl
from jax.experimental.pallas import tpu as pltpu


def _reduce_mlp_body(nnzf, vals_ref, rtw_ref, rtb_ref, w1t_ref, b1_ref,
                     w2_ref, b2_ref, att_ref, sc_ref, of_ref, acc_ref):
    j = pl.program_id(0)
    nb = pl.num_programs(0)

    @pl.when(j == 0)
    def _():
        acc_ref[...] = jnp.zeros_like(acc_ref)

    acc_ref[...] += vals_ref[...]

    @pl.when(j == nb - 1)
    def _():
        sums = jnp.sum(acc_ref[...], axis=1, keepdims=True)
        feats = rtw_ref[...] * sums + nnzf * rtb_ref[...]
        h = jnp.sum(feats * w1t_ref[...], axis=0, keepdims=True) + b1_ref[...]
        h = jnp.maximum(h, 0.0)
        logits = (jnp.sum(h * w2_ref[...], axis=1, keepdims=True)
                  + b2_ref[...])
        m = jnp.max(logits, axis=0, keepdims=True)
        e = jnp.exp(logits - m)
        att = e / jnp.sum(e, axis=0, keepdims=True)
        att_ref[...] = att
        sc_ref[...] = att * rtw_ref[...]
        of_ref[...] = att * rtb_ref[...]


def _scale_dma_body(R, NNZ, BN, nb, vals_ref, sc_ref, of_ref, out_ref,
                    buf_ref, sems):
    j = pl.program_id(0)

    def desc(bb, rr):
        return pltpu.make_async_copy(
            buf_ref.at[bb, rr],
            out_ref.at[pl.ds(rr * NNZ + j * BN, BN)],
            sems.at[bb, rr])

    for bb in range(2):
        @pl.when(j % 2 == bb)
        def _(bb=bb):
            @pl.when(j >= 2)
            def _():
                for rr in range(R):
                    desc(bb, rr).wait()

            buf_ref[bb] = vals_ref[...] * sc_ref[...] + of_ref[...]
            for rr in range(R):
                desc(bb, rr).start()

            @pl.when(j == nb - 1)
            def _():
                for rr in range(R):
                    desc(bb, rr).wait()
                if nb > 1:
                    for rr in range(R):
                        desc(1 - bb, rr).wait()


def kernel(A_indices, A_values, rt_w, rt_b, W1, b1, W2, b2):
    R, NNZ = A_values.shape
    BN = min(131072, NNZ // 8)
    nb = NNZ // BN
    H = W1.shape[0]

    idx_out = jnp.transpose(A_indices, (1, 0, 2))

    def body(*refs):
        _reduce_mlp_body(float(NNZ), *refs)

    small = lambda j: (0, 0)
    att_c, scale_c, off_c = pl.pallas_call(
        body,
        grid=(nb,),
        in_specs=[
            pl.BlockSpec((R, BN), lambda j: (0, j)),
            pl.BlockSpec((R, 1), small),
            pl.BlockSpec((R, 1), small),
            pl.BlockSpec((R, H), small),
            pl.BlockSpec((1, H), small),
            pl.BlockSpec((R, H), small),
            pl.BlockSpec((R, 1), small),
        ],
        out_specs=[pl.BlockSpec((R, 1), small)] * 3,
        out_shape=[jax.ShapeDtypeStruct((R, 1), jnp.float32)] * 3,
        scratch_shapes=[pltpu.VMEM((R, BN), jnp.float32)],
        cost_estimate=pl.CostEstimate(
            flops=R * NNZ, transcendentals=64,
            bytes_accessed=4 * R * NNZ),
    )(A_values, rt_w.reshape(R, 1), rt_b.reshape(R, 1), W1.T,
      b1.reshape(1, H), W2, b2.reshape(R, 1))

    def scale_body(*refs):
        _scale_dma_body(R, NNZ, BN, nb, *refs)

    vals_out = pl.pallas_call(
        scale_body,
        grid=(nb,),
        in_specs=[
            pl.BlockSpec((R, BN), lambda j: (0, j)),
            pl.BlockSpec((R, 1), small),
            pl.BlockSpec((R, 1), small),
        ],
        out_specs=pl.BlockSpec(memory_space=pl.ANY),
        out_shape=jax.ShapeDtypeStruct((R * NNZ,), jnp.float32),
        scratch_shapes=[
            pltpu.VMEM((2, R, BN), jnp.float32),
            pltpu.SemaphoreType.DMA((2, R)),
        ],
    )(A_values, scale_c, off_c)

    return (idx_out.reshape(2, R * NNZ), vals_out, att_c.reshape(R))


# speedup vs baseline: 1.7698x; 1.0625x over previous
import jax
import jax.numpy as jnp
from jax.experimental import pallas as pl
from jax.experimental.pallas import tpu as pltpu


def _reduce_mlp_body(nnzf, vals_ref, rtw_ref, rtb_ref, w1t_ref, b1_ref,
                     w2_ref, b2_ref, att_ref, sc_ref, of_ref, acc_ref):
    j = pl.program_id(0)
    nb = pl.num_programs(0)

    @pl.when(j == 0)
    def _():
        acc_ref[...] = jnp.zeros_like(acc_ref)

    acc_ref[...] += vals_ref[...]

    @pl.when(j == nb - 1)
    def _():
        sums = jnp.sum(acc_ref[...], axis=1, keepdims=True)
        feats = rtw_ref[...] * sums + nnzf * rtb_ref[...]
        h = jnp.sum(feats * w1t_ref[...], axis=0, keepdims=True) + b1_ref[...]
        h = jnp.maximum(h, 0.0)
        logits = (jnp.sum(h * w2_ref[...], axis=1, keepdims=True)
                  + b2_ref[...])
        m = jnp.max(logits, axis=0, keepdims=True)
        e = jnp.exp(logits - m)
        att = e / jnp.sum(e, axis=0, keepdims=True)
        att_ref[...] = att
        sc_ref[...] = att * rtw_ref[...]
        of_ref[...] = att * rtb_ref[...]


def _scale_dma_body(R, NNZ, BN, nb, vals_ref, sc_ref, of_ref, out_ref,
                    buf_ref, sems):
    j = pl.program_id(0)

    def desc(bb, rr):
        return pltpu.make_async_copy(
            buf_ref.at[bb, rr],
            out_ref.at[pl.ds(rr * NNZ + j * BN, BN)],
            sems.at[bb, rr])

    for bb in range(2):
        @pl.when(j % 2 == bb)
        def _(bb=bb):
            @pl.when(j >= 2)
            def _():
                for rr in range(R):
                    desc(bb, rr).wait()

            buf_ref[bb] = vals_ref[...] * sc_ref[...] + of_ref[...]
            for rr in range(R):
                desc(bb, rr).start()

            @pl.when(j == nb - 1)
            def _():
                for rr in range(R):
                    desc(bb, rr).wait()
                if nb > 1:
                    for rr in range(R):
                        desc(1 - bb, rr).wait()


def kernel(A_indices, A_values, rt_w, rt_b, W1, b1, W2, b2):
    R, NNZ = A_values.shape
    BN = min(262144, NNZ // 8)
    nb = NNZ // BN
    H = W1.shape[0]

    idx_out = jnp.transpose(A_indices, (1, 0, 2))

    def body(*refs):
        _reduce_mlp_body(float(NNZ), *refs)

    small = lambda j: (0, 0)
    att_c, scale_c, off_c = pl.pallas_call(
        body,
        grid=(nb,),
        in_specs=[
            pl.BlockSpec((R, BN), lambda j: (0, j)),
            pl.BlockSpec((R, 1), small),
            pl.BlockSpec((R, 1), small),
            pl.BlockSpec((R, H), small),
            pl.BlockSpec((1, H), small),
            pl.BlockSpec((R, H), small),
            pl.BlockSpec((R, 1), small),
        ],
        out_specs=[pl.BlockSpec((R, 1), small)] * 3,
        out_shape=[jax.ShapeDtypeStruct((R, 1), jnp.float32)] * 3,
        scratch_shapes=[pltpu.VMEM((R, BN), jnp.float32)],
        cost_estimate=pl.CostEstimate(
            flops=R * NNZ, transcendentals=64,
            bytes_accessed=4 * R * NNZ),
    )(A_values, rt_w.reshape(R, 1), rt_b.reshape(R, 1), W1.T,
      b1.reshape(1, H), W2, b2.reshape(R, 1))

    def scale_body(*refs):
        _scale_dma_body(R, NNZ, BN, nb, *refs)

    vals_out = pl.pallas_call(
        scale_body,
        grid=(nb,),
        in_specs=[
            pl.BlockSpec((R, BN), lambda j: (0, j)),
            pl.BlockSpec((R, 1), small),
            pl.BlockSpec((R, 1), small),
        ],
        out_specs=pl.BlockSpec(memory_space=pl.ANY),
        out_shape=jax.ShapeDtypeStruct((R * NNZ,), jnp.float32),
        scratch_shapes=[
            pltpu.VMEM((2, R, BN), jnp.float32),
            pltpu.SemaphoreType.DMA((2, R)),
        ],
    )(A_values, scale_c, off_c)

    return (idx_out.reshape(2, R * NNZ), vals_out, att_c.reshape(R))
